# baseline (speedup 1.0000x reference)
import functools

import jax
import jax.numpy as jnp
from jax import lax
from jax.experimental import pallas as pl
from jax.experimental.pallas import tpu as pltpu
from jax.experimental.pallas import tpu_sc as plsc

EPS = 1e-5

NC = 2
NS = 16
NW = NC * NS


def _sc_mesh():
    return plsc.VectorSubcoreMesh(core_axis_name="c", subcore_axis_name="s")


def _make_agg_kernel(npad, e, h, k):
    epw = e // NW
    ch = epw // k
    rpt = npad // NS
    zr = 64

    @functools.partial(
        pl.kernel,
        mesh=_sc_mesh(),
        out_type=jax.ShapeDtypeStruct((NC, npad, h), jnp.float32),
        scratch_types=[
            pltpu.VMEM((k,), jnp.int32),
            pltpu.VMEM((k,), jnp.int32),
            pltpu.VMEM((k,), jnp.int32),
            pltpu.VMEM((k,), jnp.int32),
            pltpu.VMEM((k, h), jnp.float32),
            pltpu.VMEM((k, h), jnp.float32),
            pltpu.VMEM((zr, h), jnp.float32),
            pltpu.VMEM_SHARED((npad, h), jnp.float32),
            pltpu.SemaphoreType.DMA,
            pltpu.SemaphoreType.DMA,
            pltpu.SemaphoreType.DMA,
            pltpu.SemaphoreType.DMA,
        ],
    )
    def agg_kernel(tab_hbm, src_hbm, dst_hbm, out_hbm,
                   sidx0, didx0, sidx1, didx1, rows0, rows1, zbuf,
                   acc_sh, gsem0, gsem1, ssem0, ssem1):
        cid = lax.axis_index("c")
        sid = lax.axis_index("s")
        base = (sid * NC + cid) * epw

        bufs = ((sidx0, didx0, rows0, gsem0, ssem0),
                (sidx1, didx1, rows1, gsem1, ssem1))

        zero16 = jnp.zeros((16,), jnp.float32)

        def zrow(i, _):
            def lane(j, _):
                zbuf[i, pl.ds(j * 16, 16)] = zero16
                return 0
            lax.fori_loop(0, h // 16, lane, 0)
            return 0

        lax.fori_loop(0, zr, zrow, 0)

        def zcopy(j, _):
            pltpu.sync_copy(zbuf, acc_sh.at[pl.ds(sid * rpt + j * zr, zr)])
            return 0

        lax.fori_loop(0, rpt // zr, zcopy, 0)
        plsc.subcore_barrier()

        def body(t, _):
            for p in (0, 1):
                sidx, didx, rows, gsem, ssem = bufs[p]
                is_p = t % 2 == p

                @pl.when(is_p & (t >= 2))
                def _():
                    pltpu.make_async_copy(rows, acc_sh.at[didx], ssem).wait()

                @pl.when(is_p & (t < ch))
                def _():
                    off = base + t * k
                    pltpu.sync_copy(src_hbm.at[pl.ds(off, k)], sidx)
                    pltpu.sync_copy(dst_hbm.at[pl.ds(off, k)], didx)
                    pltpu.async_copy(tab_hbm.at[sidx], rows, gsem)

                @pl.when(is_p & (t >= 1) & (t <= ch))
                def _():
                    sq, dq, rq, gq, ssq = bufs[1 - p]
                    pltpu.make_async_copy(tab_hbm.at[sq], rq, gq).wait()
                    pltpu.async_copy(rq, acc_sh.at[dq], ssq, add=True)
            return 0

        lax.fori_loop(0, ch + 2, body, 0)
        plsc.subcore_barrier()
        pltpu.sync_copy(
            acc_sh.at[pl.ds(sid * rpt, rpt)],
            out_hbm.at[cid, pl.ds(sid * rpt, rpt)],
        )

    return agg_kernel


def _edge_deg_body(ea_ref, src_ref, dst_ref, lo_ref, hi_ref, we1_ref, be1_ref,
                   esum_ref, ecnt_ref, dm_ref):
    i = pl.program_id(0)
    u = jnp.maximum(
        jnp.dot(ea_ref[...], we1_ref[...], preferred_element_type=jnp.float32)
        + be1_ref[...], 0.0)
    srow = src_ref[0]
    lo = lo_ref[:, 0:1]
    hi = hi_ref[:, 0:1]
    oh = ((srow >= lo) & (srow < hi)).astype(jnp.float32)

    drow = dst_ref[0]
    ohhi = (drow // 128
            == lax.broadcasted_iota(jnp.int32, (80, 1), 0)).astype(jnp.float32)
    ohlot = (drow % 128
             == lax.broadcasted_iota(jnp.int32, (128, 1), 0)).astype(jnp.float32)

    @pl.when(i == 0)
    def _():
        esum_ref[...] = jnp.zeros_like(esum_ref)
        ecnt_ref[...] = jnp.zeros_like(ecnt_ref)
        dm_ref[...] = jnp.zeros_like(dm_ref)

    esum_ref[...] += jnp.dot(oh, u, preferred_element_type=jnp.float32)
    ecnt_ref[...] += jnp.broadcast_to(
        jnp.sum(oh, axis=1, keepdims=True), ecnt_ref.shape)
    dm_ref[...] += lax.dot_general(
        ohhi, ohlot, (((1,), (1,)), ((), ())),
        preferred_element_type=jnp.float32)


def _k1_body(bn, x_ref, w1_ref, dm_ref, hs1_ref, dinv16_ref):
    nsub = bn // 128
    sel = (lax.broadcasted_iota(jnp.int32, (bn, nsub), 0) // 128
           == lax.broadcasted_iota(jnp.int32, (bn, nsub), 1)
           ).astype(jnp.float32)
    expand = lax.dot_general(sel, dm_ref[...], (((1,), (0,)), ((), ())),
                             preferred_element_type=jnp.float32)
    msk = (lax.broadcasted_iota(jnp.int32, (bn, 128), 0) % 128
           == lax.broadcasted_iota(jnp.int32, (bn, 128), 1))
    deg = jnp.sum(jnp.where(msk, expand, 0.0), axis=1, keepdims=True) + 1.0
    dinv = lax.rsqrt(jnp.maximum(deg, 1.0))
    hm = jnp.dot(x_ref[...], w1_ref[...], preferred_element_type=jnp.float32)
    hs1_ref[...] = hm * dinv
    dinv16_ref[...] = jnp.broadcast_to(dinv, dinv16_ref.shape)


def _pre_stats_body(ngrid, padcnt, accp_ref, hs_ref, dinv16_ref, b_ref,
                    out_ref, stats_ref):
    i = pl.program_id(0)
    dinv = dinv16_ref[:, 0:1]
    o = dinv * (accp_ref[0] + accp_ref[1] + hs_ref[...]) + b_ref[...]
    out_ref[...] = o

    @pl.when(i == 0)
    def _():
        stats_ref[...] = jnp.zeros_like(stats_ref)

    stats_ref[0:1, :] += jnp.sum(o, axis=0, keepdims=True)
    stats_ref[1:2, :] += jnp.sum(o * o, axis=0, keepdims=True)

    @pl.when(i == ngrid - 1)
    def _():
        last = o[-1:, :]
        stats_ref[0:1, :] += -float(padcnt) * last
        stats_ref[1:2, :] += -float(padcnt) * last * last


def _bn_mm_body(n, bn, o_ref, stats_ref, g_ref, bt_ref, w2_ref, dinv16_ref,
                hs2_ref):
    i = pl.program_id(0)
    mu = stats_ref[0:1, :] * (1.0 / n)
    ex2 = stats_ref[1:2, :] * (1.0 / n)
    inv = lax.rsqrt(ex2 - mu * mu + EPS)
    hcur = jnp.maximum((o_ref[...] - mu) * inv * g_ref[...] + bt_ref[...], 0.0)
    hm = jnp.dot(hcur, w2_ref[...], preferred_element_type=jnp.float32)
    rmask = (lax.broadcasted_iota(jnp.int32, (bn, 1), 0) + i * bn
             < n).astype(jnp.float32)
    hs2_ref[...] = hm * dinv16_ref[:, 0:1] * rmask


def _bn_pool_body(n, g, ngrid, o_ref, stats_ref, g_ref, bt_ref, batch_ref,
                  esum_ref, ecnt_ref, we2_ref, be2_ref, out_ref,
                  nsum_ref, ncnt_ref):
    i = pl.program_id(0)
    mu = stats_ref[0:1, :] * (1.0 / n)
    ex2 = stats_ref[1:2, :] * (1.0 / n)
    inv = lax.rsqrt(ex2 - mu * mu + EPS)
    h2 = jnp.maximum((o_ref[...] - mu) * inv * g_ref[...] + bt_ref[...], 0.0)
    brow = batch_ref[0]
    gids = lax.broadcasted_iota(jnp.int32, (g, 1), 0)
    oh = (brow == gids).astype(jnp.float32)

    @pl.when(i == 0)
    def _():
        nsum_ref[...] = jnp.zeros_like(nsum_ref)
        ncnt_ref[...] = jnp.zeros_like(ncnt_ref)

    nsum_ref[...] += jnp.dot(oh, h2, preferred_element_type=jnp.float32)
    ncnt_ref[...] += jnp.broadcast_to(
        jnp.sum(oh, axis=1, keepdims=True), ncnt_ref.shape)

    @pl.when(i == ngrid - 1)
    def _():
        ecnt = ecnt_ref[...]
        edge_part = (
            jnp.dot(esum_ref[...], we2_ref[...],
                    preferred_element_type=jnp.float32)
            + ecnt * be2_ref[...]) / jnp.maximum(ecnt, 1.0)
        node_part = nsum_ref[...] / jnp.maximum(ncnt_ref[...], 1.0)
        out_ref[...] = node_part + edge_part


def kernel(x, edge_index, edge_attr, batch_idx,
           W1, b1, g1, bt1, W2, b2, g2, bt2,
           We1, be1, We2, be2):
    n, df = x.shape
    e = edge_index.shape[1]
    de = edge_attr.shape[1]
    h = W1.shape[1]
    g = 64
    f32 = jnp.float32

    npad = 10240
    padcnt = npad - n
    bn = 2048
    ngrid = npad // bn
    be = 2000
    egrid = e // be

    src = edge_index[0].astype(jnp.int32)
    dst = edge_index[1].astype(jnp.int32)
    src3d = src.reshape(egrid, 1, be)
    dst3d = dst.reshape(egrid, 1, be)

    k_sc = 128
    epad = NW * k_sc * (-(-e // (NW * k_sc)))
    srcp = jnp.full((epad,), npad - 1, jnp.int32).at[:e].set(src)
    dstp = jnp.full((epad,), npad - 1, jnp.int32).at[:e].set(dst)

    starts = jnp.searchsorted(
        batch_idx.astype(jnp.int32), jnp.arange(g + 1, dtype=jnp.int32)
    ).astype(jnp.int32)
    lo_b = jnp.broadcast_to(starts[:g][:, None], (g, h)).astype(jnp.int32)
    hi_b = jnp.broadcast_to(starts[1:][:, None], (g, h)).astype(jnp.int32)

    xpad = jnp.zeros((npad, df), f32).at[:n].set(x)
    bpad = jnp.full((npad,), g, jnp.int32).at[:n].set(batch_idx.astype(jnp.int32))
    batch3d = bpad.reshape(ngrid, 1, bn)

    esum, ecnt, degmat = pl.pallas_call(
        _edge_deg_body,
        grid=(egrid,),
        in_specs=[
            pl.BlockSpec((be, de), lambda i: (i, 0)),
            pl.BlockSpec((1, 1, be), lambda i: (i, 0, 0)),
            pl.BlockSpec((1, 1, be), lambda i: (i, 0, 0)),
            pl.BlockSpec((g, h), lambda i: (0, 0)),
            pl.BlockSpec((g, h), lambda i: (0, 0)),
            pl.BlockSpec((de, h), lambda i: (0, 0)),
            pl.BlockSpec((1, h), lambda i: (0, 0)),
        ],
        out_specs=[
            pl.BlockSpec((g, h), lambda i: (0, 0)),
            pl.BlockSpec((g, h), lambda i: (0, 0)),
            pl.BlockSpec((80, 128), lambda i: (0, 0)),
        ],
        out_shape=[
            jax.ShapeDtypeStruct((g, h), f32),
            jax.ShapeDtypeStruct((g, h), f32),
            jax.ShapeDtypeStruct((80, 128), f32),
        ],
    )(edge_attr, src3d, dst3d, lo_b, hi_b, We1, be1.reshape(1, h))

    hs1, dinv16 = pl.pallas_call(
        functools.partial(_k1_body, bn),
        grid=(ngrid,),
        in_specs=[
            pl.BlockSpec((bn, df), lambda i: (i, 0)),
            pl.BlockSpec((df, h), lambda i: (0, 0)),
            pl.BlockSpec((bn // 128, 128), lambda i: (i, 0)),
        ],
        out_specs=[
            pl.BlockSpec((bn, h), lambda i: (i, 0)),
            pl.BlockSpec((bn, 16), lambda i: (i, 0)),
        ],
        out_shape=[
            jax.ShapeDtypeStruct((npad, h), f32),
            jax.ShapeDtypeStruct((npad, 16), f32),
        ],
    )(xpad, W1, degmat)

    agg = _make_agg_kernel(npad, epad, h, k_sc)

    def pre_stats(accp, hs, bias):
        return pl.pallas_call(
            functools.partial(_pre_stats_body, ngrid, padcnt),
            grid=(ngrid,),
            in_specs=[
                pl.BlockSpec((NC, bn, h), lambda i: (0, i, 0)),
                pl.BlockSpec((bn, h), lambda i: (i, 0)),
                pl.BlockSpec((bn, 16), lambda i: (i, 0)),
                pl.BlockSpec((1, h), lambda i: (0, 0)),
            ],
            out_specs=[
                pl.BlockSpec((bn, h), lambda i: (i, 0)),
                pl.BlockSpec((8, h), lambda i: (0, 0)),
            ],
            out_shape=[
                jax.ShapeDtypeStruct((npad, h), f32),
                jax.ShapeDtypeStruct((8, h), f32),
            ],
        )(accp, hs, dinv16, bias.reshape(1, h))

    acc1 = agg(hs1, srcp, dstp)
    out1_pre, stats1 = pre_stats(acc1, hs1, b1)

    hs2 = pl.pallas_call(
        functools.partial(_bn_mm_body, n, bn),
        grid=(ngrid,),
        in_specs=[
            pl.BlockSpec((bn, h), lambda i: (i, 0)),
            pl.BlockSpec((8, h), lambda i: (0, 0)),
            pl.BlockSpec((1, h), lambda i: (0, 0)),
            pl.BlockSpec((1, h), lambda i: (0, 0)),
            pl.BlockSpec((h, h), lambda i: (0, 0)),
            pl.BlockSpec((bn, 16), lambda i: (i, 0)),
        ],
        out_specs=pl.BlockSpec((bn, h), lambda i: (i, 0)),
        out_shape=jax.ShapeDtypeStruct((npad, h), f32),
    )(out1_pre, stats1, g1.reshape(1, h), bt1.reshape(1, h), W2, dinv16)

    acc2 = agg(hs2, srcp, dstp)
    out2_pre, stats2 = pre_stats(acc2, hs2, b2)

    out = pl.pallas_call(
        functools.partial(_bn_pool_body, n, g, ngrid),
        grid=(ngrid,),
        in_specs=[
            pl.BlockSpec((bn, h), lambda i: (i, 0)),
            pl.BlockSpec((8, h), lambda i: (0, 0)),
            pl.BlockSpec((1, h), lambda i: (0, 0)),
            pl.BlockSpec((1, h), lambda i: (0, 0)),
            pl.BlockSpec((1, 1, bn), lambda i: (i, 0, 0)),
            pl.BlockSpec((g, h), lambda i: (0, 0)),
            pl.BlockSpec((g, h), lambda i: (0, 0)),
            pl.BlockSpec((h, h), lambda i: (0, 0)),
            pl.BlockSpec((1, h), lambda i: (0, 0)),
        ],
        out_specs=pl.BlockSpec((g, h), lambda i: (0, 0)),
        out_shape=jax.ShapeDtypeStruct((g, h), f32),
        scratch_shapes=[
            pltpu.VMEM((g, h), f32),
            pltpu.VMEM((g, h), f32),
        ],
    )(out2_pre, stats2, g2.reshape(1, h), bt2.reshape(1, h), batch3d,
      esum, ecnt, We2, be2.reshape(1, h))

    return out


# speedup vs baseline: 1.1118x; 1.1118x over previous
import functools

import jax
import jax.numpy as jnp
from jax import lax
from jax.experimental import pallas as pl
from jax.experimental.pallas import tpu as pltpu
from jax.experimental.pallas import tpu_sc as plsc

EPS = 1e-5

NC = 2
NS = 16
NW = NC * NS


def _sc_mesh():
    return plsc.VectorSubcoreMesh(core_axis_name="c", subcore_axis_name="s")


def _make_agg_kernel(npad, e, h, k, ch0_frac=0.5):
    cht = e // (NS * k)
    ch0 = int(round(cht * ch0_frac))
    ch1 = cht - ch0
    rpt = npad // NS
    zr = 64

    @functools.partial(
        pl.kernel,
        mesh=_sc_mesh(),
        out_type=jax.ShapeDtypeStruct((NC, npad, h), jnp.float32),
        scratch_types=[
            pltpu.VMEM((k,), jnp.int32),
            pltpu.VMEM((k,), jnp.int32),
            pltpu.VMEM((k,), jnp.int32),
            pltpu.VMEM((k,), jnp.int32),
            pltpu.VMEM((k, h), jnp.float32),
            pltpu.VMEM((k, h), jnp.float32),
            pltpu.VMEM((zr, h), jnp.float32),
            pltpu.VMEM_SHARED((npad, h), jnp.float32),
            pltpu.SemaphoreType.DMA,
            pltpu.SemaphoreType.DMA,
            pltpu.SemaphoreType.DMA,
            pltpu.SemaphoreType.DMA,
        ],
    )
    def agg_kernel(tab_hbm, src_hbm, dst_hbm, out_hbm,
                   sidx0, didx0, sidx1, didx1, rows0, rows1, zbuf,
                   acc_sh, gsem0, gsem1, ssem0, ssem1):
        cid = lax.axis_index("c")
        sid = lax.axis_index("s")
        ch = jnp.where(cid == 0, ch0, ch1)
        base = jnp.where(cid == 0, sid * ch0, NS * ch0 + sid * ch1) * k

        bufs = ((sidx0, didx0, rows0, gsem0, ssem0),
                (sidx1, didx1, rows1, gsem1, ssem1))

        zero16 = jnp.zeros((16,), jnp.float32)

        def zrow(i, _):
            def lane(j, _):
                zbuf[i, pl.ds(j * 16, 16)] = zero16
                return 0
            lax.fori_loop(0, h // 16, lane, 0)
            return 0

        lax.fori_loop(0, zr, zrow, 0)

        def zcopy(j, _):
            pltpu.sync_copy(zbuf, acc_sh.at[pl.ds(sid * rpt + j * zr, zr)])
            return 0

        lax.fori_loop(0, rpt // zr, zcopy, 0)
        plsc.subcore_barrier()

        def body(t, _):
            for p in (0, 1):
                sidx, didx, rows, gsem, ssem = bufs[p]
                is_p = t % 2 == p

                @pl.when(is_p & (t >= 2))
                def _():
                    pltpu.make_async_copy(rows, acc_sh.at[didx], ssem).wait()

                @pl.when(is_p & (t < ch))
                def _():
                    off = base + t * k
                    pltpu.sync_copy(src_hbm.at[pl.ds(off, k)], sidx)
                    pltpu.sync_copy(dst_hbm.at[pl.ds(off, k)], didx)
                    pltpu.async_copy(tab_hbm.at[sidx], rows, gsem)

                @pl.when(is_p & (t >= 1) & (t <= ch))
                def _():
                    sq, dq, rq, gq, ssq = bufs[1 - p]
                    pltpu.make_async_copy(tab_hbm.at[sq], rq, gq).wait()
                    pltpu.async_copy(rq, acc_sh.at[dq], ssq, add=True)
            return 0

        lax.fori_loop(0, ch + 2, body, 0)
        plsc.subcore_barrier()
        pltpu.sync_copy(
            acc_sh.at[pl.ds(sid * rpt, rpt)],
            out_hbm.at[cid, pl.ds(sid * rpt, rpt)],
        )

    return agg_kernel


def _edge_deg_body(ea_ref, src_ref, dst_ref, lo_ref, hi_ref, we1_ref, be1_ref,
                   esum_ref, ecnt_ref, dm_ref):
    i = pl.program_id(0)
    u = jnp.maximum(
        jnp.dot(ea_ref[...], we1_ref[...], preferred_element_type=jnp.float32)
        + be1_ref[...], 0.0)
    srow = src_ref[0]
    lo = lo_ref[:, 0:1]
    hi = hi_ref[:, 0:1]
    oh = ((srow >= lo) & (srow < hi)).astype(jnp.float32)

    drow = dst_ref[0]
    ohhi = (drow // 128
            == lax.broadcasted_iota(jnp.int32, (80, 1), 0)).astype(jnp.float32)
    ohlot = (drow % 128
             == lax.broadcasted_iota(jnp.int32, (128, 1), 0)).astype(jnp.float32)

    @pl.when(i == 0)
    def _():
        esum_ref[...] = jnp.zeros_like(esum_ref)
        ecnt_ref[...] = jnp.zeros_like(ecnt_ref)
        dm_ref[...] = jnp.zeros_like(dm_ref)

    esum_ref[...] += jnp.dot(oh, u, preferred_element_type=jnp.float32)
    ecnt_ref[...] += jnp.broadcast_to(
        jnp.sum(oh, axis=1, keepdims=True), ecnt_ref.shape)
    dm_ref[...] += lax.dot_general(
        ohhi, ohlot, (((1,), (1,)), ((), ())),
        preferred_element_type=jnp.float32)


def _k1_body(bn, x_ref, w1_ref, dm_ref, hs1_ref, dinv16_ref):
    nsub = bn // 128
    sel = (lax.broadcasted_iota(jnp.int32, (bn, nsub), 0) // 128
           == lax.broadcasted_iota(jnp.int32, (bn, nsub), 1)
           ).astype(jnp.float32)
    expand = lax.dot_general(sel, dm_ref[...], (((1,), (0,)), ((), ())),
                             preferred_element_type=jnp.float32)
    msk = (lax.broadcasted_iota(jnp.int32, (bn, 128), 0) % 128
           == lax.broadcasted_iota(jnp.int32, (bn, 128), 1))
    deg = jnp.sum(jnp.where(msk, expand, 0.0), axis=1, keepdims=True) + 1.0
    dinv = lax.rsqrt(jnp.maximum(deg, 1.0))
    hm = jnp.dot(x_ref[...], w1_ref[...], preferred_element_type=jnp.float32)
    hs1_ref[...] = hm * dinv
    dinv16_ref[...] = jnp.broadcast_to(dinv, dinv16_ref.shape)


def _pre_stats_body(ngrid, padcnt, accp_ref, hs_ref, dinv16_ref, b_ref,
                    out_ref, stats_ref):
    i = pl.program_id(0)
    dinv = dinv16_ref[:, 0:1]
    o = dinv * (accp_ref[0] + accp_ref[1] + hs_ref[...]) + b_ref[...]
    out_ref[...] = o

    @pl.when(i == 0)
    def _():
        stats_ref[...] = jnp.zeros_like(stats_ref)

    stats_ref[0:1, :] += jnp.sum(o, axis=0, keepdims=True)
    stats_ref[1:2, :] += jnp.sum(o * o, axis=0, keepdims=True)

    @pl.when(i == ngrid - 1)
    def _():
        last = o[-1:, :]
        stats_ref[0:1, :] += -float(padcnt) * last
        stats_ref[1:2, :] += -float(padcnt) * last * last


def _bn_mm_body(n, bn, o_ref, stats_ref, g_ref, bt_ref, w2_ref, dinv16_ref,
                hs2_ref):
    i = pl.program_id(0)
    mu = stats_ref[0:1, :] * (1.0 / n)
    ex2 = stats_ref[1:2, :] * (1.0 / n)
    inv = lax.rsqrt(ex2 - mu * mu + EPS)
    hcur = jnp.maximum((o_ref[...] - mu) * inv * g_ref[...] + bt_ref[...], 0.0)
    hm = jnp.dot(hcur, w2_ref[...], preferred_element_type=jnp.float32)
    rmask = (lax.broadcasted_iota(jnp.int32, (bn, 1), 0) + i * bn
             < n).astype(jnp.float32)
    hs2_ref[...] = hm * dinv16_ref[:, 0:1] * rmask


def _bn_pool_body(n, g, ngrid, o_ref, stats_ref, g_ref, bt_ref, batch_ref,
                  esum_ref, ecnt_ref, we2_ref, be2_ref, out_ref,
                  nsum_ref, ncnt_ref):
    i = pl.program_id(0)
    mu = stats_ref[0:1, :] * (1.0 / n)
    ex2 = stats_ref[1:2, :] * (1.0 / n)
    inv = lax.rsqrt(ex2 - mu * mu + EPS)
    h2 = jnp.maximum((o_ref[...] - mu) * inv * g_ref[...] + bt_ref[...], 0.0)
    brow = batch_ref[0]
    gids = lax.broadcasted_iota(jnp.int32, (g, 1), 0)
    oh = (brow == gids).astype(jnp.float32)

    @pl.when(i == 0)
    def _():
        nsum_ref[...] = jnp.zeros_like(nsum_ref)
        ncnt_ref[...] = jnp.zeros_like(ncnt_ref)

    nsum_ref[...] += jnp.dot(oh, h2, preferred_element_type=jnp.float32)
    ncnt_ref[...] += jnp.broadcast_to(
        jnp.sum(oh, axis=1, keepdims=True), ncnt_ref.shape)

    @pl.when(i == ngrid - 1)
    def _():
        ecnt = ecnt_ref[...]
        edge_part = (
            jnp.dot(esum_ref[...], we2_ref[...],
                    preferred_element_type=jnp.float32)
            + ecnt * be2_ref[...]) / jnp.maximum(ecnt, 1.0)
        node_part = nsum_ref[...] / jnp.maximum(ncnt_ref[...], 1.0)
        out_ref[...] = node_part + edge_part


def kernel(x, edge_index, edge_attr, batch_idx,
           W1, b1, g1, bt1, W2, b2, g2, bt2,
           We1, be1, We2, be2):
    n, df = x.shape
    e = edge_index.shape[1]
    de = edge_attr.shape[1]
    h = W1.shape[1]
    g = 64
    f32 = jnp.float32

    npad = 10240
    padcnt = npad - n
    bn = 2048
    ngrid = npad // bn
    be = 2000
    egrid = e // be

    src = edge_index[0].astype(jnp.int32)
    dst = edge_index[1].astype(jnp.int32)
    src3d = src.reshape(egrid, 1, be)
    dst3d = dst.reshape(egrid, 1, be)

    k_sc = 128
    epad = NW * k_sc * (-(-e // (NW * k_sc)))
    srcp = jnp.full((epad,), npad - 1, jnp.int32).at[:e].set(src)
    dstp = jnp.full((epad,), npad - 1, jnp.int32).at[:e].set(dst)

    starts = jnp.searchsorted(
        batch_idx.astype(jnp.int32), jnp.arange(g + 1, dtype=jnp.int32)
    ).astype(jnp.int32)
    lo_b = jnp.broadcast_to(starts[:g][:, None], (g, h)).astype(jnp.int32)
    hi_b = jnp.broadcast_to(starts[1:][:, None], (g, h)).astype(jnp.int32)

    xpad = jnp.zeros((npad, df), f32).at[:n].set(x)
    bpad = jnp.full((npad,), g, jnp.int32).at[:n].set(batch_idx.astype(jnp.int32))
    batch3d = bpad.reshape(ngrid, 1, bn)

    esum, ecnt, degmat = pl.pallas_call(
        _edge_deg_body,
        grid=(egrid,),
        in_specs=[
            pl.BlockSpec((be, de), lambda i: (i, 0)),
            pl.BlockSpec((1, 1, be), lambda i: (i, 0, 0)),
            pl.BlockSpec((1, 1, be), lambda i: (i, 0, 0)),
            pl.BlockSpec((g, h), lambda i: (0, 0)),
            pl.BlockSpec((g, h), lambda i: (0, 0)),
            pl.BlockSpec((de, h), lambda i: (0, 0)),
            pl.BlockSpec((1, h), lambda i: (0, 0)),
        ],
        out_specs=[
            pl.BlockSpec((g, h), lambda i: (0, 0)),
            pl.BlockSpec((g, h), lambda i: (0, 0)),
            pl.BlockSpec((80, 128), lambda i: (0, 0)),
        ],
        out_shape=[
            jax.ShapeDtypeStruct((g, h), f32),
            jax.ShapeDtypeStruct((g, h), f32),
            jax.ShapeDtypeStruct((80, 128), f32),
        ],
    )(edge_attr, src3d, dst3d, lo_b, hi_b, We1, be1.reshape(1, h))

    hs1, dinv16 = pl.pallas_call(
        functools.partial(_k1_body, bn),
        grid=(ngrid,),
        in_specs=[
            pl.BlockSpec((bn, df), lambda i: (i, 0)),
            pl.BlockSpec((df, h), lambda i: (0, 0)),
            pl.BlockSpec((bn // 128, 128), lambda i: (i, 0)),
        ],
        out_specs=[
            pl.BlockSpec((bn, h), lambda i: (i, 0)),
            pl.BlockSpec((bn, 16), lambda i: (i, 0)),
        ],
        out_shape=[
            jax.ShapeDtypeStruct((npad, h), f32),
            jax.ShapeDtypeStruct((npad, 16), f32),
        ],
    )(xpad, W1, degmat)

    agg = _make_agg_kernel(npad, epad, h, k_sc, ch0_frac=0.7)

    def pre_stats(accp, hs, bias):
        return pl.pallas_call(
            functools.partial(_pre_stats_body, ngrid, padcnt),
            grid=(ngrid,),
            in_specs=[
                pl.BlockSpec((NC, bn, h), lambda i: (0, i, 0)),
                pl.BlockSpec((bn, h), lambda i: (i, 0)),
                pl.BlockSpec((bn, 16), lambda i: (i, 0)),
                pl.BlockSpec((1, h), lambda i: (0, 0)),
            ],
            out_specs=[
                pl.BlockSpec((bn, h), lambda i: (i, 0)),
                pl.BlockSpec((8, h), lambda i: (0, 0)),
            ],
            out_shape=[
                jax.ShapeDtypeStruct((npad, h), f32),
                jax.ShapeDtypeStruct((8, h), f32),
            ],
        )(accp, hs, dinv16, bias.reshape(1, h))

    acc1 = agg(hs1, srcp, dstp)
    out1_pre, stats1 = pre_stats(acc1, hs1, b1)

    hs2 = pl.pallas_call(
        functools.partial(_bn_mm_body, n, bn),
        grid=(ngrid,),
        in_specs=[
            pl.BlockSpec((bn, h), lambda i: (i, 0)),
            pl.BlockSpec((8, h), lambda i: (0, 0)),
            pl.BlockSpec((1, h), lambda i: (0, 0)),
            pl.BlockSpec((1, h), lambda i: (0, 0)),
            pl.BlockSpec((h, h), lambda i: (0, 0)),
            pl.BlockSpec((bn, 16), lambda i: (i, 0)),
        ],
        out_specs=pl.BlockSpec((bn, h), lambda i: (i, 0)),
        out_shape=jax.ShapeDtypeStruct((npad, h), f32),
    )(out1_pre, stats1, g1.reshape(1, h), bt1.reshape(1, h), W2, dinv16)

    acc2 = agg(hs2, srcp, dstp)
    out2_pre, stats2 = pre_stats(acc2, hs2, b2)

    out = pl.pallas_call(
        functools.partial(_bn_pool_body, n, g, ngrid),
        grid=(ngrid,),
        in_specs=[
            pl.BlockSpec((bn, h), lambda i: (i, 0)),
            pl.BlockSpec((8, h), lambda i: (0, 0)),
            pl.BlockSpec((1, h), lambda i: (0, 0)),
            pl.BlockSpec((1, h), lambda i: (0, 0)),
            pl.BlockSpec((1, 1, bn), lambda i: (i, 0, 0)),
            pl.BlockSpec((g, h), lambda i: (0, 0)),
            pl.BlockSpec((g, h), lambda i: (0, 0)),
            pl.BlockSpec((h, h), lambda i: (0, 0)),
            pl.BlockSpec((1, h), lambda i: (0, 0)),
        ],
        out_specs=pl.BlockSpec((g, h), lambda i: (0, 0)),
        out_shape=jax.ShapeDtypeStruct((g, h), f32),
        scratch_shapes=[
            pltpu.VMEM((g, h), f32),
            pltpu.VMEM((g, h), f32),
        ],
    )(out2_pre, stats2, g2.reshape(1, h), bt2.reshape(1, h), batch3d,
      esum, ecnt, We2, be2.reshape(1, h))

    return out


# speedup vs baseline: 1.1950x; 1.0749x over previous
import functools

import jax
import jax.numpy as jnp
from jax import lax
from jax.experimental import pallas as pl
from jax.experimental.pallas import tpu as pltpu
from jax.experimental.pallas import tpu_sc as plsc

EPS = 1e-5

NC = 2
NS = 16
NW = NC * NS


def _sc_mesh():
    return plsc.VectorSubcoreMesh(core_axis_name="c", subcore_axis_name="s")


def _make_agg_kernel(npad, e, h, k, ch0_frac=0.5):
    cht = e // (NS * k)
    ch0 = int(round(cht * ch0_frac))
    ch1 = cht - ch0
    rpt = npad // NS
    zr = 64

    @functools.partial(
        pl.kernel,
        mesh=_sc_mesh(),
        out_type=jax.ShapeDtypeStruct((NC, npad, h), jnp.float32),
        scratch_types=[
            pltpu.VMEM((k,), jnp.int32),
            pltpu.VMEM((k,), jnp.int32),
            pltpu.VMEM((k,), jnp.int32),
            pltpu.VMEM((k,), jnp.int32),
            pltpu.VMEM((k, h), jnp.float32),
            pltpu.VMEM((k, h), jnp.float32),
            pltpu.VMEM((zr, h), jnp.float32),
            pltpu.VMEM_SHARED((npad, h), jnp.float32),
            pltpu.SemaphoreType.DMA,
            pltpu.SemaphoreType.DMA,
            pltpu.SemaphoreType.DMA,
            pltpu.SemaphoreType.DMA,
        ],
    )
    def agg_kernel(tab_hbm, src_hbm, dst_hbm, out_hbm,
                   sidx0, didx0, sidx1, didx1, rows0, rows1, zbuf,
                   acc_sh, gsem0, gsem1, ssem0, ssem1):
        cid = lax.axis_index("c")
        sid = lax.axis_index("s")
        ch = jnp.where(cid == 0, ch0, ch1)
        base = jnp.where(cid == 0, sid * ch0, NS * ch0 + sid * ch1) * k

        bufs = ((sidx0, didx0, rows0, gsem0, ssem0),
                (sidx1, didx1, rows1, gsem1, ssem1))

        zero16 = jnp.zeros((16,), jnp.float32)

        def zrow(i, _):
            def lane(j, _):
                zbuf[i, pl.ds(j * 16, 16)] = zero16
                return 0
            lax.fori_loop(0, h // 16, lane, 0)
            return 0

        lax.fori_loop(0, zr, zrow, 0)

        def zcopy(j, _):
            pltpu.sync_copy(zbuf, acc_sh.at[pl.ds(sid * rpt + j * zr, zr)])
            return 0

        lax.fori_loop(0, rpt // zr, zcopy, 0)
        plsc.subcore_barrier()

        def body(t, _):
            for p in (0, 1):
                sidx, didx, rows, gsem, ssem = bufs[p]
                is_p = t % 2 == p

                @pl.when(is_p & (t >= 2))
                def _():
                    pltpu.make_async_copy(rows, acc_sh.at[didx], ssem).wait()

                @pl.when(is_p & (t < ch))
                def _():
                    off = base + t * k
                    pltpu.sync_copy(src_hbm.at[pl.ds(off, k)], sidx)
                    pltpu.sync_copy(dst_hbm.at[pl.ds(off, k)], didx)
                    pltpu.async_copy(tab_hbm.at[sidx], rows, gsem)

                @pl.when(is_p & (t >= 1) & (t <= ch))
                def _():
                    sq, dq, rq, gq, ssq = bufs[1 - p]
                    pltpu.make_async_copy(tab_hbm.at[sq], rq, gq).wait()
                    pltpu.async_copy(rq, acc_sh.at[dq], ssq, add=True)
            return 0

        lax.fori_loop(0, ch + 2, body, 0)
        plsc.subcore_barrier()
        pltpu.sync_copy(
            acc_sh.at[pl.ds(sid * rpt, rpt)],
            out_hbm.at[cid, pl.ds(sid * rpt, rpt)],
        )

    return agg_kernel


def _edge_deg_body(ea_ref, src_ref, dst_ref, lo_ref, hi_ref, we1_ref, be1_ref,
                   esum_ref, ecnt_ref, dm_ref):
    i = pl.program_id(0)
    u = jnp.maximum(
        jnp.dot(ea_ref[...], we1_ref[...], preferred_element_type=jnp.float32)
        + be1_ref[...], 0.0)
    srow = src_ref[0]
    lo = lo_ref[:, 0:1]
    hi = hi_ref[:, 0:1]
    oh = ((srow >= lo) & (srow < hi)).astype(jnp.float32)

    drow = dst_ref[0]
    ohhi = (drow // 128
            == lax.broadcasted_iota(jnp.int32, (80, 1), 0)).astype(jnp.float32)
    ohlot = (drow % 128
             == lax.broadcasted_iota(jnp.int32, (128, 1), 0)).astype(jnp.float32)

    @pl.when(i == 0)
    def _():
        esum_ref[...] = jnp.zeros_like(esum_ref)
        ecnt_ref[...] = jnp.zeros_like(ecnt_ref)
        dm_ref[...] = jnp.zeros_like(dm_ref)

    esum_ref[...] += jnp.dot(oh, u, preferred_element_type=jnp.float32)
    ecnt_ref[...] += jnp.broadcast_to(
        jnp.sum(oh, axis=1, keepdims=True), ecnt_ref.shape)
    dm_ref[...] += lax.dot_general(
        ohhi, ohlot, (((1,), (1,)), ((), ())),
        preferred_element_type=jnp.float32)


def _k1_body(bn, x_ref, w1_ref, dm_ref, hs1_ref, dinv16_ref):
    nsub = bn // 128
    sel = (lax.broadcasted_iota(jnp.int32, (bn, nsub), 0) // 128
           == lax.broadcasted_iota(jnp.int32, (bn, nsub), 1)
           ).astype(jnp.float32)
    expand = lax.dot_general(sel, dm_ref[...], (((1,), (0,)), ((), ())),
                             preferred_element_type=jnp.float32)
    msk = (lax.broadcasted_iota(jnp.int32, (bn, 128), 0) % 128
           == lax.broadcasted_iota(jnp.int32, (bn, 128), 1))
    deg = jnp.sum(jnp.where(msk, expand, 0.0), axis=1, keepdims=True) + 1.0
    dinv = lax.rsqrt(jnp.maximum(deg, 1.0))
    hm = jnp.dot(x_ref[...], w1_ref[...], preferred_element_type=jnp.float32)
    hs1_ref[...] = hm * dinv
    dinv16_ref[...] = jnp.broadcast_to(dinv, dinv16_ref.shape)


def _pre_stats_body(ngrid, padcnt, accp_ref, hs_ref, dinv16_ref, b_ref,
                    out_ref, stats_ref):
    i = pl.program_id(0)
    dinv = dinv16_ref[:, 0:1]
    o = dinv * (accp_ref[0] + accp_ref[1] + hs_ref[...]) + b_ref[...]
    out_ref[...] = o

    @pl.when(i == 0)
    def _():
        stats_ref[...] = jnp.zeros_like(stats_ref)

    stats_ref[0:1, :] += jnp.sum(o, axis=0, keepdims=True)
    stats_ref[1:2, :] += jnp.sum(o * o, axis=0, keepdims=True)

    @pl.when(i == ngrid - 1)
    def _():
        last = o[-1:, :]
        stats_ref[0:1, :] += -float(padcnt) * last
        stats_ref[1:2, :] += -float(padcnt) * last * last


def _bn_mm_body(n, bn, o_ref, stats_ref, g_ref, bt_ref, w2_ref, dinv16_ref,
                hs2_ref):
    i = pl.program_id(0)
    mu = stats_ref[0:1, :] * (1.0 / n)
    ex2 = stats_ref[1:2, :] * (1.0 / n)
    inv = lax.rsqrt(ex2 - mu * mu + EPS)
    hcur = jnp.maximum((o_ref[...] - mu) * inv * g_ref[...] + bt_ref[...], 0.0)
    hm = jnp.dot(hcur, w2_ref[...], preferred_element_type=jnp.float32)
    rmask = (lax.broadcasted_iota(jnp.int32, (bn, 1), 0) + i * bn
             < n).astype(jnp.float32)
    hs2_ref[...] = hm * dinv16_ref[:, 0:1] * rmask


def _bn_pool_body(n, g, ngrid, o_ref, stats_ref, g_ref, bt_ref, batch_ref,
                  esum_ref, ecnt_ref, we2_ref, be2_ref, out_ref,
                  nsum_ref, ncnt_ref):
    i = pl.program_id(0)
    mu = stats_ref[0:1, :] * (1.0 / n)
    ex2 = stats_ref[1:2, :] * (1.0 / n)
    inv = lax.rsqrt(ex2 - mu * mu + EPS)
    h2 = jnp.maximum((o_ref[...] - mu) * inv * g_ref[...] + bt_ref[...], 0.0)
    brow = batch_ref[0]
    gids = lax.broadcasted_iota(jnp.int32, (g, 1), 0)
    oh = (brow == gids).astype(jnp.float32)

    @pl.when(i == 0)
    def _():
        nsum_ref[...] = jnp.zeros_like(nsum_ref)
        ncnt_ref[...] = jnp.zeros_like(ncnt_ref)

    nsum_ref[...] += jnp.dot(oh, h2, preferred_element_type=jnp.float32)
    ncnt_ref[...] += jnp.broadcast_to(
        jnp.sum(oh, axis=1, keepdims=True), ncnt_ref.shape)

    @pl.when(i == ngrid - 1)
    def _():
        ecnt = ecnt_ref[...]
        edge_part = (
            jnp.dot(esum_ref[...], we2_ref[...],
                    preferred_element_type=jnp.float32)
            + ecnt * be2_ref[...]) / jnp.maximum(ecnt, 1.0)
        node_part = nsum_ref[...] / jnp.maximum(ncnt_ref[...], 1.0)
        out_ref[...] = node_part + edge_part


def kernel(x, edge_index, edge_attr, batch_idx,
           W1, b1, g1, bt1, W2, b2, g2, bt2,
           We1, be1, We2, be2):
    n, df = x.shape
    e = edge_index.shape[1]
    de = edge_attr.shape[1]
    h = W1.shape[1]
    g = 64
    f32 = jnp.float32

    npad = 10240
    padcnt = npad - n
    bn = 2048
    ngrid = npad // bn
    be = 2000
    egrid = e // be

    src = edge_index[0].astype(jnp.int32)
    dst = edge_index[1].astype(jnp.int32)
    src3d = src.reshape(egrid, 1, be)
    dst3d = dst.reshape(egrid, 1, be)

    k_sc = 128
    epad = NW * k_sc * (-(-e // (NW * k_sc)))
    srcp = jnp.full((epad,), npad - 1, jnp.int32).at[:e].set(src)
    dstp = jnp.full((epad,), npad - 1, jnp.int32).at[:e].set(dst)

    starts = jnp.searchsorted(
        batch_idx.astype(jnp.int32), jnp.arange(g + 1, dtype=jnp.int32)
    ).astype(jnp.int32)
    lo_b = jnp.broadcast_to(starts[:g][:, None], (g, h)).astype(jnp.int32)
    hi_b = jnp.broadcast_to(starts[1:][:, None], (g, h)).astype(jnp.int32)

    xpad = jnp.zeros((npad, df), f32).at[:n].set(x)
    bpad = jnp.full((npad,), g, jnp.int32).at[:n].set(batch_idx.astype(jnp.int32))
    batch3d = bpad.reshape(ngrid, 1, bn)

    esum, ecnt, degmat = pl.pallas_call(
        _edge_deg_body,
        grid=(egrid,),
        in_specs=[
            pl.BlockSpec((be, de), lambda i: (i, 0)),
            pl.BlockSpec((1, 1, be), lambda i: (i, 0, 0)),
            pl.BlockSpec((1, 1, be), lambda i: (i, 0, 0)),
            pl.BlockSpec((g, h), lambda i: (0, 0)),
            pl.BlockSpec((g, h), lambda i: (0, 0)),
            pl.BlockSpec((de, h), lambda i: (0, 0)),
            pl.BlockSpec((1, h), lambda i: (0, 0)),
        ],
        out_specs=[
            pl.BlockSpec((g, h), lambda i: (0, 0)),
            pl.BlockSpec((g, h), lambda i: (0, 0)),
            pl.BlockSpec((80, 128), lambda i: (0, 0)),
        ],
        out_shape=[
            jax.ShapeDtypeStruct((g, h), f32),
            jax.ShapeDtypeStruct((g, h), f32),
            jax.ShapeDtypeStruct((80, 128), f32),
        ],
    )(edge_attr, src3d, dst3d, lo_b, hi_b, We1, be1.reshape(1, h))

    hs1, dinv16 = pl.pallas_call(
        functools.partial(_k1_body, bn),
        grid=(ngrid,),
        in_specs=[
            pl.BlockSpec((bn, df), lambda i: (i, 0)),
            pl.BlockSpec((df, h), lambda i: (0, 0)),
            pl.BlockSpec((bn // 128, 128), lambda i: (i, 0)),
        ],
        out_specs=[
            pl.BlockSpec((bn, h), lambda i: (i, 0)),
            pl.BlockSpec((bn, 16), lambda i: (i, 0)),
        ],
        out_shape=[
            jax.ShapeDtypeStruct((npad, h), f32),
            jax.ShapeDtypeStruct((npad, 16), f32),
        ],
    )(xpad, W1, degmat)

    agg = _make_agg_kernel(npad, epad, h, k_sc, ch0_frac=0.875)

    def pre_stats(accp, hs, bias):
        return pl.pallas_call(
            functools.partial(_pre_stats_body, ngrid, padcnt),
            grid=(ngrid,),
            in_specs=[
                pl.BlockSpec((NC, bn, h), lambda i: (0, i, 0)),
                pl.BlockSpec((bn, h), lambda i: (i, 0)),
                pl.BlockSpec((bn, 16), lambda i: (i, 0)),
                pl.BlockSpec((1, h), lambda i: (0, 0)),
            ],
            out_specs=[
                pl.BlockSpec((bn, h), lambda i: (i, 0)),
                pl.BlockSpec((8, h), lambda i: (0, 0)),
            ],
            out_shape=[
                jax.ShapeDtypeStruct((npad, h), f32),
                jax.ShapeDtypeStruct((8, h), f32),
            ],
        )(accp, hs, dinv16, bias.reshape(1, h))

    acc1 = agg(hs1, srcp, dstp)
    out1_pre, stats1 = pre_stats(acc1, hs1, b1)

    hs2 = pl.pallas_call(
        functools.partial(_bn_mm_body, n, bn),
        grid=(ngrid,),
        in_specs=[
            pl.BlockSpec((bn, h), lambda i: (i, 0)),
            pl.BlockSpec((8, h), lambda i: (0, 0)),
            pl.BlockSpec((1, h), lambda i: (0, 0)),
            pl.BlockSpec((1, h), lambda i: (0, 0)),
            pl.BlockSpec((h, h), lambda i: (0, 0)),
            pl.BlockSpec((bn, 16), lambda i: (i, 0)),
        ],
        out_specs=pl.BlockSpec((bn, h), lambda i: (i, 0)),
        out_shape=jax.ShapeDtypeStruct((npad, h), f32),
    )(out1_pre, stats1, g1.reshape(1, h), bt1.reshape(1, h), W2, dinv16)

    acc2 = agg(hs2, srcp, dstp)
    out2_pre, stats2 = pre_stats(acc2, hs2, b2)

    out = pl.pallas_call(
        functools.partial(_bn_pool_body, n, g, ngrid),
        grid=(ngrid,),
        in_specs=[
            pl.BlockSpec((bn, h), lambda i: (i, 0)),
            pl.BlockSpec((8, h), lambda i: (0, 0)),
            pl.BlockSpec((1, h), lambda i: (0, 0)),
            pl.BlockSpec((1, h), lambda i: (0, 0)),
            pl.BlockSpec((1, 1, bn), lambda i: (i, 0, 0)),
            pl.BlockSpec((g, h), lambda i: (0, 0)),
            pl.BlockSpec((g, h), lambda i: (0, 0)),
            pl.BlockSpec((h, h), lambda i: (0, 0)),
            pl.BlockSpec((1, h), lambda i: (0, 0)),
        ],
        out_specs=pl.BlockSpec((g, h), lambda i: (0, 0)),
        out_shape=jax.ShapeDtypeStruct((g, h), f32),
        scratch_shapes=[
            pltpu.VMEM((g, h), f32),
            pltpu.VMEM((g, h), f32),
        ],
    )(out2_pre, stats2, g2.reshape(1, h), bt2.reshape(1, h), batch3d,
      esum, ecnt, We2, be2.reshape(1, h))

    return out


# speedup vs baseline: 1.2892x; 1.0788x over previous
import functools

import jax
import jax.numpy as jnp
from jax import lax
from jax.experimental import pallas as pl
from jax.experimental.pallas import tpu as pltpu
from jax.experimental.pallas import tpu_sc as plsc

EPS = 1e-5

NC = 2
NS = 16
NW = NC * NS


def _sc_mesh():
    return plsc.VectorSubcoreMesh(core_axis_name="c", subcore_axis_name="s")


def _make_agg_kernel(npad, e, h, k, ch0_frac=0.5):
    cht = e // (NS * k)
    ch0 = int(round(cht * ch0_frac))
    ch1 = cht - ch0
    rpt = npad // NS
    zr = 64

    @functools.partial(
        pl.kernel,
        mesh=_sc_mesh(),
        out_type=jax.ShapeDtypeStruct((NC, npad, h), jnp.float32),
        scratch_types=[
            pltpu.VMEM((k,), jnp.int32),
            pltpu.VMEM((k,), jnp.int32),
            pltpu.VMEM((k,), jnp.int32),
            pltpu.VMEM((k,), jnp.int32),
            pltpu.VMEM((k, h), jnp.float32),
            pltpu.VMEM((k, h), jnp.float32),
            pltpu.VMEM((zr, h), jnp.float32),
            pltpu.VMEM_SHARED((npad, h), jnp.float32),
            pltpu.SemaphoreType.DMA,
            pltpu.SemaphoreType.DMA,
            pltpu.SemaphoreType.DMA,
            pltpu.SemaphoreType.DMA,
        ],
    )
    def agg_kernel(tab_hbm, src_hbm, dst_hbm, out_hbm,
                   sidx0, didx0, sidx1, didx1, rows0, rows1, zbuf,
                   acc_sh, gsem0, gsem1, ssem0, ssem1):
        cid = lax.axis_index("c")
        sid = lax.axis_index("s")
        ch = jnp.where(cid == 0, ch0, ch1)
        base = jnp.where(cid == 0, sid * ch0, NS * ch0 + sid * ch1) * k

        bufs = ((sidx0, didx0, rows0, gsem0, ssem0),
                (sidx1, didx1, rows1, gsem1, ssem1))

        zero16 = jnp.zeros((16,), jnp.float32)

        def zrow(i, _):
            def lane(j, _):
                zbuf[i, pl.ds(j * 16, 16)] = zero16
                return 0
            lax.fori_loop(0, h // 16, lane, 0)
            return 0

        lax.fori_loop(0, zr, zrow, 0)

        def zcopy(j, _):
            pltpu.sync_copy(zbuf, acc_sh.at[pl.ds(sid * rpt + j * zr, zr)])
            return 0

        lax.fori_loop(0, rpt // zr, zcopy, 0)
        plsc.subcore_barrier()

        def body(t, _):
            for p in (0, 1):
                sidx, didx, rows, gsem, ssem = bufs[p]
                is_p = t % 2 == p

                @pl.when(is_p & (t >= 2))
                def _():
                    pltpu.make_async_copy(rows, acc_sh.at[didx], ssem).wait()

                @pl.when(is_p & (t < ch))
                def _():
                    off = base + t * k
                    pltpu.sync_copy(src_hbm.at[pl.ds(off, k)], sidx)
                    pltpu.sync_copy(dst_hbm.at[pl.ds(off, k)], didx)
                    pltpu.async_copy(tab_hbm.at[sidx], rows, gsem)

                @pl.when(is_p & (t >= 1) & (t <= ch))
                def _():
                    sq, dq, rq, gq, ssq = bufs[1 - p]
                    pltpu.make_async_copy(tab_hbm.at[sq], rq, gq).wait()
                    pltpu.async_copy(rq, acc_sh.at[dq], ssq, add=True)
            return 0

        lax.fori_loop(0, ch + 2, body, 0)
        plsc.subcore_barrier()
        pltpu.sync_copy(
            acc_sh.at[pl.ds(sid * rpt, rpt)],
            out_hbm.at[cid, pl.ds(sid * rpt, rpt)],
        )

    return agg_kernel


def _edge_deg_body(ea_ref, src_ref, dst_ref, lo_ref, hi_ref, we1_ref, be1_ref,
                   esum_ref, ecnt_ref, dm_ref):
    i = pl.program_id(0)
    bf16 = jnp.bfloat16
    u = jnp.maximum(
        jnp.dot(ea_ref[...], we1_ref[...], preferred_element_type=jnp.float32)
        + be1_ref[...], 0.0)
    srow = src_ref[0]
    lo = lo_ref[:, 0:1]
    hi = hi_ref[:, 0:1]
    oh = ((srow >= lo) & (srow < hi)).astype(bf16)

    drow = dst_ref[0]
    ohhi = (drow // 128
            == lax.broadcasted_iota(jnp.int32, (80, 1), 0)).astype(bf16)
    ohlot = (drow % 128
             == lax.broadcasted_iota(jnp.int32, (128, 1), 0)).astype(bf16)

    @pl.when(i == 0)
    def _():
        esum_ref[...] = jnp.zeros_like(esum_ref)
        ecnt_ref[...] = jnp.zeros_like(ecnt_ref)
        dm_ref[...] = jnp.zeros_like(dm_ref)

    esum_ref[...] += jnp.dot(oh, u.astype(bf16),
                             preferred_element_type=jnp.float32)
    ecnt_ref[...] += jnp.broadcast_to(
        jnp.sum(oh.astype(jnp.float32), axis=1, keepdims=True),
        ecnt_ref.shape)
    dm_ref[...] += lax.dot_general(
        ohhi, ohlot, (((1,), (1,)), ((), ())),
        preferred_element_type=jnp.float32)


def _k1_body(bn, x_ref, w1_ref, dm_ref, hs1_ref, dinv16_ref):
    nsub = bn // 128
    sel = (lax.broadcasted_iota(jnp.int32, (bn, nsub), 0) // 128
           == lax.broadcasted_iota(jnp.int32, (bn, nsub), 1)
           ).astype(jnp.float32)
    expand = lax.dot_general(sel, dm_ref[...], (((1,), (0,)), ((), ())),
                             preferred_element_type=jnp.float32)
    msk = (lax.broadcasted_iota(jnp.int32, (bn, 128), 0) % 128
           == lax.broadcasted_iota(jnp.int32, (bn, 128), 1))
    deg = jnp.sum(jnp.where(msk, expand, 0.0), axis=1, keepdims=True) + 1.0
    dinv = lax.rsqrt(jnp.maximum(deg, 1.0))
    hm = jnp.dot(x_ref[...], w1_ref[...], preferred_element_type=jnp.float32)
    hs1_ref[...] = hm * dinv
    dinv16_ref[...] = jnp.broadcast_to(dinv, dinv16_ref.shape)


def _pre_stats_body(ngrid, padcnt, accp_ref, hs_ref, dinv16_ref, b_ref,
                    out_ref, stats_ref):
    i = pl.program_id(0)
    dinv = dinv16_ref[:, 0:1]
    o = dinv * (accp_ref[0] + accp_ref[1] + hs_ref[...]) + b_ref[...]
    out_ref[...] = o

    @pl.when(i == 0)
    def _():
        stats_ref[...] = jnp.zeros_like(stats_ref)

    stats_ref[0:1, :] += jnp.sum(o, axis=0, keepdims=True)
    stats_ref[1:2, :] += jnp.sum(o * o, axis=0, keepdims=True)

    @pl.when(i == ngrid - 1)
    def _():
        last = o[-1:, :]
        stats_ref[0:1, :] += -float(padcnt) * last
        stats_ref[1:2, :] += -float(padcnt) * last * last


def _bn_mm_body(n, bn, o_ref, stats_ref, g_ref, bt_ref, w2_ref, dinv16_ref,
                hs2_ref):
    i = pl.program_id(0)
    mu = stats_ref[0:1, :] * (1.0 / n)
    ex2 = stats_ref[1:2, :] * (1.0 / n)
    inv = lax.rsqrt(ex2 - mu * mu + EPS)
    hcur = jnp.maximum((o_ref[...] - mu) * inv * g_ref[...] + bt_ref[...], 0.0)
    hm = jnp.dot(hcur, w2_ref[...], preferred_element_type=jnp.float32)
    rmask = (lax.broadcasted_iota(jnp.int32, (bn, 1), 0) + i * bn
             < n).astype(jnp.float32)
    hs2_ref[...] = hm * dinv16_ref[:, 0:1] * rmask


def _bn_pool_body(n, g, ngrid, o_ref, stats_ref, g_ref, bt_ref, batch_ref,
                  esum_ref, ecnt_ref, we2_ref, be2_ref, out_ref,
                  nsum_ref, ncnt_ref):
    i = pl.program_id(0)
    mu = stats_ref[0:1, :] * (1.0 / n)
    ex2 = stats_ref[1:2, :] * (1.0 / n)
    inv = lax.rsqrt(ex2 - mu * mu + EPS)
    h2 = jnp.maximum((o_ref[...] - mu) * inv * g_ref[...] + bt_ref[...], 0.0)
    brow = batch_ref[0]
    gids = lax.broadcasted_iota(jnp.int32, (g, 1), 0)
    oh = (brow == gids).astype(jnp.float32)

    @pl.when(i == 0)
    def _():
        nsum_ref[...] = jnp.zeros_like(nsum_ref)
        ncnt_ref[...] = jnp.zeros_like(ncnt_ref)

    nsum_ref[...] += jnp.dot(oh, h2, preferred_element_type=jnp.float32)
    ncnt_ref[...] += jnp.broadcast_to(
        jnp.sum(oh, axis=1, keepdims=True), ncnt_ref.shape)

    @pl.when(i == ngrid - 1)
    def _():
        ecnt = ecnt_ref[...]
        edge_part = (
            jnp.dot(esum_ref[...], we2_ref[...],
                    preferred_element_type=jnp.float32)
            + ecnt * be2_ref[...]) / jnp.maximum(ecnt, 1.0)
        node_part = nsum_ref[...] / jnp.maximum(ncnt_ref[...], 1.0)
        out_ref[...] = node_part + edge_part


def kernel(x, edge_index, edge_attr, batch_idx,
           W1, b1, g1, bt1, W2, b2, g2, bt2,
           We1, be1, We2, be2):
    n, df = x.shape
    e = edge_index.shape[1]
    de = edge_attr.shape[1]
    h = W1.shape[1]
    g = 64
    f32 = jnp.float32

    npad = 10240
    padcnt = npad - n
    bn = 2048
    ngrid = npad // bn
    be = 8000
    egrid = e // be

    src = edge_index[0].astype(jnp.int32)
    dst = edge_index[1].astype(jnp.int32)
    src3d = src.reshape(egrid, 1, be)
    dst3d = dst.reshape(egrid, 1, be)

    k_sc = 128
    epad = NW * k_sc * (-(-e // (NW * k_sc)))
    srcp = jnp.full((epad,), npad - 1, jnp.int32).at[:e].set(src)
    dstp = jnp.full((epad,), npad - 1, jnp.int32).at[:e].set(dst)

    starts = jnp.searchsorted(
        batch_idx.astype(jnp.int32), jnp.arange(g + 1, dtype=jnp.int32)
    ).astype(jnp.int32)
    lo_b = jnp.broadcast_to(starts[:g][:, None], (g, h)).astype(jnp.int32)
    hi_b = jnp.broadcast_to(starts[1:][:, None], (g, h)).astype(jnp.int32)

    xpad = jnp.zeros((npad, df), f32).at[:n].set(x)
    bpad = jnp.full((npad,), g, jnp.int32).at[:n].set(batch_idx.astype(jnp.int32))
    batch3d = bpad.reshape(ngrid, 1, bn)

    esum, ecnt, degmat = pl.pallas_call(
        _edge_deg_body,
        grid=(egrid,),
        in_specs=[
            pl.BlockSpec((be, de), lambda i: (i, 0)),
            pl.BlockSpec((1, 1, be), lambda i: (i, 0, 0)),
            pl.BlockSpec((1, 1, be), lambda i: (i, 0, 0)),
            pl.BlockSpec((g, h), lambda i: (0, 0)),
            pl.BlockSpec((g, h), lambda i: (0, 0)),
            pl.BlockSpec((de, h), lambda i: (0, 0)),
            pl.BlockSpec((1, h), lambda i: (0, 0)),
        ],
        out_specs=[
            pl.BlockSpec((g, h), lambda i: (0, 0)),
            pl.BlockSpec((g, h), lambda i: (0, 0)),
            pl.BlockSpec((80, 128), lambda i: (0, 0)),
        ],
        out_shape=[
            jax.ShapeDtypeStruct((g, h), f32),
            jax.ShapeDtypeStruct((g, h), f32),
            jax.ShapeDtypeStruct((80, 128), f32),
        ],
    )(edge_attr.astype(jnp.bfloat16), src3d, dst3d, lo_b, hi_b,
      We1.astype(jnp.bfloat16), be1.reshape(1, h))

    hs1, dinv16 = pl.pallas_call(
        functools.partial(_k1_body, bn),
        grid=(ngrid,),
        in_specs=[
            pl.BlockSpec((bn, df), lambda i: (i, 0)),
            pl.BlockSpec((df, h), lambda i: (0, 0)),
            pl.BlockSpec((bn // 128, 128), lambda i: (i, 0)),
        ],
        out_specs=[
            pl.BlockSpec((bn, h), lambda i: (i, 0)),
            pl.BlockSpec((bn, 16), lambda i: (i, 0)),
        ],
        out_shape=[
            jax.ShapeDtypeStruct((npad, h), f32),
            jax.ShapeDtypeStruct((npad, 16), f32),
        ],
    )(xpad, W1, degmat)

    agg = _make_agg_kernel(npad, epad, h, k_sc, ch0_frac=0.875)

    def pre_stats(accp, hs, bias):
        return pl.pallas_call(
            functools.partial(_pre_stats_body, ngrid, padcnt),
            grid=(ngrid,),
            in_specs=[
                pl.BlockSpec((NC, bn, h), lambda i: (0, i, 0)),
                pl.BlockSpec((bn, h), lambda i: (i, 0)),
                pl.BlockSpec((bn, 16), lambda i: (i, 0)),
                pl.BlockSpec((1, h), lambda i: (0, 0)),
            ],
            out_specs=[
                pl.BlockSpec((bn, h), lambda i: (i, 0)),
                pl.BlockSpec((8, h), lambda i: (0, 0)),
            ],
            out_shape=[
                jax.ShapeDtypeStruct((npad, h), f32),
                jax.ShapeDtypeStruct((8, h), f32),
            ],
        )(accp, hs, dinv16, bias.reshape(1, h))

    acc1 = agg(hs1, srcp, dstp)
    out1_pre, stats1 = pre_stats(acc1, hs1, b1)

    hs2 = pl.pallas_call(
        functools.partial(_bn_mm_body, n, bn),
        grid=(ngrid,),
        in_specs=[
            pl.BlockSpec((bn, h), lambda i: (i, 0)),
            pl.BlockSpec((8, h), lambda i: (0, 0)),
            pl.BlockSpec((1, h), lambda i: (0, 0)),
            pl.BlockSpec((1, h), lambda i: (0, 0)),
            pl.BlockSpec((h, h), lambda i: (0, 0)),
            pl.BlockSpec((bn, 16), lambda i: (i, 0)),
        ],
        out_specs=pl.BlockSpec((bn, h), lambda i: (i, 0)),
        out_shape=jax.ShapeDtypeStruct((npad, h), f32),
    )(out1_pre, stats1, g1.reshape(1, h), bt1.reshape(1, h), W2, dinv16)

    acc2 = agg(hs2, srcp, dstp)
    out2_pre, stats2 = pre_stats(acc2, hs2, b2)

    out = pl.pallas_call(
        functools.partial(_bn_pool_body, n, g, ngrid),
        grid=(ngrid,),
        in_specs=[
            pl.BlockSpec((bn, h), lambda i: (i, 0)),
            pl.BlockSpec((8, h), lambda i: (0, 0)),
            pl.BlockSpec((1, h), lambda i: (0, 0)),
            pl.BlockSpec((1, h), lambda i: (0, 0)),
            pl.BlockSpec((1, 1, bn), lambda i: (i, 0, 0)),
            pl.BlockSpec((g, h), lambda i: (0, 0)),
            pl.BlockSpec((g, h), lambda i: (0, 0)),
            pl.BlockSpec((h, h), lambda i: (0, 0)),
            pl.BlockSpec((1, h), lambda i: (0, 0)),
        ],
        out_specs=pl.BlockSpec((g, h), lambda i: (0, 0)),
        out_shape=jax.ShapeDtypeStruct((g, h), f32),
        scratch_shapes=[
            pltpu.VMEM((g, h), f32),
            pltpu.VMEM((g, h), f32),
        ],
    )(out2_pre, stats2, g2.reshape(1, h), bt2.reshape(1, h), batch3d,
      esum, ecnt, We2, be2.reshape(1, h))

    return out


# speedup vs baseline: 1.3176x; 1.0220x over previous
import functools

import jax
import jax.numpy as jnp
from jax import lax
from jax.experimental import pallas as pl
from jax.experimental.pallas import tpu as pltpu
from jax.experimental.pallas import tpu_sc as plsc

EPS = 1e-5

NC = 2
NS = 16
NW = NC * NS


def _sc_mesh():
    return plsc.VectorSubcoreMesh(core_axis_name="c", subcore_axis_name="s")


def _make_agg_kernel(npad, e, h, k, ch0_frac=0.5):
    cht = e // (NS * k)
    ch0 = int(round(cht * ch0_frac))
    ch1 = cht - ch0
    rpt = npad // NS
    zr = 64

    @functools.partial(
        pl.kernel,
        mesh=_sc_mesh(),
        out_type=jax.ShapeDtypeStruct((NC, npad, h), jnp.float32),
        scratch_types=[
            pltpu.VMEM((k,), jnp.int32),
            pltpu.VMEM((k,), jnp.int32),
            pltpu.VMEM((k,), jnp.int32),
            pltpu.VMEM((k,), jnp.int32),
            pltpu.VMEM((k, h), jnp.float32),
            pltpu.VMEM((k, h), jnp.float32),
            pltpu.VMEM((zr, h), jnp.float32),
            pltpu.VMEM_SHARED((npad, h), jnp.float32),
            pltpu.SemaphoreType.DMA,
            pltpu.SemaphoreType.DMA,
            pltpu.SemaphoreType.DMA,
            pltpu.SemaphoreType.DMA,
        ],
    )
    def agg_kernel(tab_hbm, src_hbm, dst_hbm, out_hbm,
                   sidx0, didx0, sidx1, didx1, rows0, rows1, zbuf,
                   acc_sh, gsem0, gsem1, ssem0, ssem1):
        cid = lax.axis_index("c")
        sid = lax.axis_index("s")
        ch = jnp.where(cid == 0, ch0, ch1)
        base = jnp.where(cid == 0, sid * ch0, NS * ch0 + sid * ch1) * k

        bufs = ((sidx0, didx0, rows0, gsem0, ssem0),
                (sidx1, didx1, rows1, gsem1, ssem1))

        zero16 = jnp.zeros((16,), jnp.float32)

        def zrow(i, _):
            def lane(j, _):
                zbuf[i, pl.ds(j * 16, 16)] = zero16
                return 0
            lax.fori_loop(0, h // 16, lane, 0)
            return 0

        lax.fori_loop(0, zr, zrow, 0)

        def zcopy(j, _):
            pltpu.sync_copy(zbuf, acc_sh.at[pl.ds(sid * rpt + j * zr, zr)])
            return 0

        lax.fori_loop(0, rpt // zr, zcopy, 0)
        plsc.subcore_barrier()

        def body(t, _):
            for p in (0, 1):
                sidx, didx, rows, gsem, ssem = bufs[p]
                is_p = t % 2 == p

                @pl.when(is_p & (t >= 2))
                def _():
                    pltpu.make_async_copy(rows, acc_sh.at[didx], ssem).wait()

                @pl.when(is_p & (t < ch))
                def _():
                    off = base + t * k
                    pltpu.sync_copy(src_hbm.at[pl.ds(off, k)], sidx)
                    pltpu.sync_copy(dst_hbm.at[pl.ds(off, k)], didx)
                    pltpu.async_copy(tab_hbm.at[sidx], rows, gsem)

                @pl.when(is_p & (t >= 1) & (t <= ch))
                def _():
                    sq, dq, rq, gq, ssq = bufs[1 - p]
                    pltpu.make_async_copy(tab_hbm.at[sq], rq, gq).wait()
                    pltpu.async_copy(rq, acc_sh.at[dq], ssq, add=True)
            return 0

        lax.fori_loop(0, ch + 2, body, 0)
        plsc.subcore_barrier()
        pltpu.sync_copy(
            acc_sh.at[pl.ds(sid * rpt, rpt)],
            out_hbm.at[cid, pl.ds(sid * rpt, rpt)],
        )

    return agg_kernel


def _edge_deg_body(ea_ref, src_ref, dst_ref, lo_ref, hi_ref, we1_ref, be1_ref,
                   esum_ref, ecnt_ref, dm_ref):
    i = pl.program_id(0)
    bf16 = jnp.bfloat16
    u = jnp.maximum(
        jnp.dot(ea_ref[...], we1_ref[...], preferred_element_type=jnp.float32)
        + be1_ref[...], 0.0)
    srow = src_ref[0]
    lo = lo_ref[:, 0:1]
    hi = hi_ref[:, 0:1]
    oh = ((srow >= lo) & (srow < hi)).astype(bf16)

    drow = dst_ref[0]
    ohhi = (drow // 128
            == lax.broadcasted_iota(jnp.int32, (80, 1), 0)).astype(bf16)
    ohlot = (drow % 128
             == lax.broadcasted_iota(jnp.int32, (128, 1), 0)).astype(bf16)

    @pl.when(i == 0)
    def _():
        esum_ref[...] = jnp.zeros_like(esum_ref)
        ecnt_ref[...] = jnp.zeros_like(ecnt_ref)
        dm_ref[...] = jnp.zeros_like(dm_ref)

    esum_ref[...] += jnp.dot(oh, u.astype(bf16),
                             preferred_element_type=jnp.float32)
    ecnt_ref[...] += jnp.broadcast_to(
        jnp.sum(oh.astype(jnp.float32), axis=1, keepdims=True),
        ecnt_ref.shape)
    dm_ref[...] += lax.dot_general(
        ohhi, ohlot, (((1,), (1,)), ((), ())),
        preferred_element_type=jnp.float32)


def _k1_body(bn, x_ref, w1_ref, dm_ref, hs1_ref, dinv16_ref):
    nsub = bn // 128
    sel = (lax.broadcasted_iota(jnp.int32, (bn, nsub), 0) // 128
           == lax.broadcasted_iota(jnp.int32, (bn, nsub), 1)
           ).astype(jnp.float32)
    expand = lax.dot_general(sel, dm_ref[...], (((1,), (0,)), ((), ())),
                             preferred_element_type=jnp.float32)
    msk = (lax.broadcasted_iota(jnp.int32, (bn, 128), 0) % 128
           == lax.broadcasted_iota(jnp.int32, (bn, 128), 1))
    deg = jnp.sum(jnp.where(msk, expand, 0.0), axis=1, keepdims=True) + 1.0
    dinv = lax.rsqrt(jnp.maximum(deg, 1.0))
    hm = jnp.dot(x_ref[...], w1_ref[...], preferred_element_type=jnp.float32)
    hs1_ref[...] = hm * dinv
    dinv16_ref[...] = jnp.broadcast_to(dinv, dinv16_ref.shape)


def _stats_phase(ngrid, padcnt, bn, accp_ref, hs_ref, dinv16_ref, b_ref,
                 o_scr, stats_ref):
    i = pl.program_id(1)
    dinv = dinv16_ref[:, 0:1]
    o = dinv * (accp_ref[0] + accp_ref[1] + hs_ref[...]) + b_ref[...]
    o_scr[pl.ds(i * bn, bn), :] = o

    @pl.when(i == 0)
    def _():
        stats_ref[...] = jnp.zeros_like(stats_ref)

    stats_ref[0:1, :] += jnp.sum(o, axis=0, keepdims=True)
    stats_ref[1:2, :] += jnp.sum(o * o, axis=0, keepdims=True)

    @pl.when(i == ngrid - 1)
    def _():
        last = o[-1:, :]
        stats_ref[0:1, :] += -float(padcnt) * last
        stats_ref[1:2, :] += -float(padcnt) * last * last


def _bn_from_stats(n, stats_ref):
    mu = stats_ref[0:1, :] * (1.0 / n)
    ex2 = stats_ref[1:2, :] * (1.0 / n)
    inv = lax.rsqrt(ex2 - mu * mu + EPS)
    return mu, inv


def _layer_mid_body(n, bn, ngrid, padcnt,
                    accp_ref, hs_ref, dinv16_ref, b_ref, g_ref, bt_ref,
                    w2_ref, hs2_ref, o_scr, stats_scr):
    j = pl.program_id(0)
    i = pl.program_id(1)

    @pl.when(j == 0)
    def _():
        _stats_phase(ngrid, padcnt, bn, accp_ref, hs_ref, dinv16_ref, b_ref,
                     o_scr, stats_scr)

    @pl.when(j == 1)
    def _():
        mu, inv = _bn_from_stats(n, stats_scr)
        o = o_scr[pl.ds(i * bn, bn), :]
        hcur = jnp.maximum((o - mu) * inv * g_ref[...] + bt_ref[...], 0.0)
        hm = jnp.dot(hcur, w2_ref[...], preferred_element_type=jnp.float32)
        rmask = (lax.broadcasted_iota(jnp.int32, (bn, 1), 0) + i * bn
                 < n).astype(jnp.float32)
        hs2_ref[...] = hm * dinv16_ref[:, 0:1] * rmask


def _layer_final_body(n, g, bn, ngrid, padcnt,
                      accp_ref, hs_ref, dinv16_ref, b_ref, g_ref, bt_ref,
                      batch_ref, esum_ref, ecnt_ref, we2_ref, be2_ref,
                      out_ref, o_scr, stats_scr, nsum_ref, ncnt_ref):
    j = pl.program_id(0)
    i = pl.program_id(1)

    @pl.when(j == 0)
    def _():
        _stats_phase(ngrid, padcnt, bn, accp_ref, hs_ref, dinv16_ref, b_ref,
                     o_scr, stats_scr)

    @pl.when(j == 1)
    def _():
        mu, inv = _bn_from_stats(n, stats_scr)
        o = o_scr[pl.ds(i * bn, bn), :]
        h2 = jnp.maximum((o - mu) * inv * g_ref[...] + bt_ref[...], 0.0)
        brow = batch_ref[0]
        gids = lax.broadcasted_iota(jnp.int32, (g, 1), 0)
        oh = (brow == gids).astype(jnp.float32)

        @pl.when(i == 0)
        def _():
            nsum_ref[...] = jnp.zeros_like(nsum_ref)
            ncnt_ref[...] = jnp.zeros_like(ncnt_ref)

        nsum_ref[...] += jnp.dot(oh, h2, preferred_element_type=jnp.float32)
        ncnt_ref[...] += jnp.broadcast_to(
            jnp.sum(oh, axis=1, keepdims=True), ncnt_ref.shape)

        @pl.when(i == ngrid - 1)
        def _():
            ecnt = ecnt_ref[...]
            edge_part = (
                jnp.dot(esum_ref[...], we2_ref[...],
                        preferred_element_type=jnp.float32)
                + ecnt * be2_ref[...]) / jnp.maximum(ecnt, 1.0)
            node_part = nsum_ref[...] / jnp.maximum(ncnt_ref[...], 1.0)
            out_ref[...] = node_part + edge_part


def kernel(x, edge_index, edge_attr, batch_idx,
           W1, b1, g1, bt1, W2, b2, g2, bt2,
           We1, be1, We2, be2):
    n, df = x.shape
    e = edge_index.shape[1]
    de = edge_attr.shape[1]
    h = W1.shape[1]
    g = 64
    f32 = jnp.float32

    npad = 10240
    padcnt = npad - n
    bn = 2048
    ngrid = npad // bn
    be = 8000
    egrid = e // be

    src = edge_index[0].astype(jnp.int32)
    dst = edge_index[1].astype(jnp.int32)
    src3d = src.reshape(egrid, 1, be)
    dst3d = dst.reshape(egrid, 1, be)

    k_sc = 128
    epad = NW * k_sc * (-(-e // (NW * k_sc)))
    srcp = jnp.full((epad,), npad - 1, jnp.int32).at[:e].set(src)
    dstp = jnp.full((epad,), npad - 1, jnp.int32).at[:e].set(dst)

    starts = jnp.searchsorted(
        batch_idx.astype(jnp.int32), jnp.arange(g + 1, dtype=jnp.int32)
    ).astype(jnp.int32)
    lo_b = jnp.broadcast_to(starts[:g][:, None], (g, h)).astype(jnp.int32)
    hi_b = jnp.broadcast_to(starts[1:][:, None], (g, h)).astype(jnp.int32)

    xpad = jnp.zeros((npad, df), f32).at[:n].set(x)
    bpad = jnp.full((npad,), g, jnp.int32).at[:n].set(batch_idx.astype(jnp.int32))
    batch3d = bpad.reshape(ngrid, 1, bn)

    esum, ecnt, degmat = pl.pallas_call(
        _edge_deg_body,
        grid=(egrid,),
        in_specs=[
            pl.BlockSpec((be, de), lambda i: (i, 0)),
            pl.BlockSpec((1, 1, be), lambda i: (i, 0, 0)),
            pl.BlockSpec((1, 1, be), lambda i: (i, 0, 0)),
            pl.BlockSpec((g, h), lambda i: (0, 0)),
            pl.BlockSpec((g, h), lambda i: (0, 0)),
            pl.BlockSpec((de, h), lambda i: (0, 0)),
            pl.BlockSpec((1, h), lambda i: (0, 0)),
        ],
        out_specs=[
            pl.BlockSpec((g, h), lambda i: (0, 0)),
            pl.BlockSpec((g, h), lambda i: (0, 0)),
            pl.BlockSpec((80, 128), lambda i: (0, 0)),
        ],
        out_shape=[
            jax.ShapeDtypeStruct((g, h), f32),
            jax.ShapeDtypeStruct((g, h), f32),
            jax.ShapeDtypeStruct((80, 128), f32),
        ],
    )(edge_attr.astype(jnp.bfloat16), src3d, dst3d, lo_b, hi_b,
      We1.astype(jnp.bfloat16), be1.reshape(1, h))

    hs1, dinv16 = pl.pallas_call(
        functools.partial(_k1_body, bn),
        grid=(ngrid,),
        in_specs=[
            pl.BlockSpec((bn, df), lambda i: (i, 0)),
            pl.BlockSpec((df, h), lambda i: (0, 0)),
            pl.BlockSpec((bn // 128, 128), lambda i: (i, 0)),
        ],
        out_specs=[
            pl.BlockSpec((bn, h), lambda i: (i, 0)),
            pl.BlockSpec((bn, 16), lambda i: (i, 0)),
        ],
        out_shape=[
            jax.ShapeDtypeStruct((npad, h), f32),
            jax.ShapeDtypeStruct((npad, 16), f32),
        ],
    )(xpad, W1, degmat)

    agg = _make_agg_kernel(npad, epad, h, k_sc, ch0_frac=0.875)

    acc1 = agg(hs1, srcp, dstp)
    hs2 = pl.pallas_call(
        functools.partial(_layer_mid_body, n, bn, ngrid, padcnt),
        grid=(2, ngrid),
        in_specs=[
            pl.BlockSpec((NC, bn, h), lambda j, i: (0, i * (1 - j), 0)),
            pl.BlockSpec((bn, h), lambda j, i: (i * (1 - j), 0)),
            pl.BlockSpec((bn, 16), lambda j, i: (i, 0)),
            pl.BlockSpec((1, h), lambda j, i: (0, 0)),
            pl.BlockSpec((1, h), lambda j, i: (0, 0)),
            pl.BlockSpec((1, h), lambda j, i: (0, 0)),
            pl.BlockSpec((h, h), lambda j, i: (0, 0)),
        ],
        out_specs=pl.BlockSpec((bn, h), lambda j, i: (i * j, 0)),
        out_shape=jax.ShapeDtypeStruct((npad, h), f32),
        scratch_shapes=[
            pltpu.VMEM((npad, h), f32),
            pltpu.VMEM((8, h), f32),
        ],
    )(acc1, hs1, dinv16, b1.reshape(1, h), g1.reshape(1, h),
      bt1.reshape(1, h), W2)

    acc2 = agg(hs2, srcp, dstp)
    out = pl.pallas_call(
        functools.partial(_layer_final_body, n, g, bn, ngrid, padcnt),
        grid=(2, ngrid),
        in_specs=[
            pl.BlockSpec((NC, bn, h), lambda j, i: (0, i * (1 - j), 0)),
            pl.BlockSpec((bn, h), lambda j, i: (i * (1 - j), 0)),
            pl.BlockSpec((bn, 16), lambda j, i: (i, 0)),
            pl.BlockSpec((1, h), lambda j, i: (0, 0)),
            pl.BlockSpec((1, h), lambda j, i: (0, 0)),
            pl.BlockSpec((1, h), lambda j, i: (0, 0)),
            pl.BlockSpec((1, 1, bn), lambda j, i: (i, 0, 0)),
            pl.BlockSpec((g, h), lambda j, i: (0, 0)),
            pl.BlockSpec((g, h), lambda j, i: (0, 0)),
            pl.BlockSpec((h, h), lambda j, i: (0, 0)),
            pl.BlockSpec((1, h), lambda j, i: (0, 0)),
        ],
        out_specs=pl.BlockSpec((g, h), lambda j, i: (0, 0)),
        out_shape=jax.ShapeDtypeStruct((g, h), f32),
        scratch_shapes=[
            pltpu.VMEM((npad, h), f32),
            pltpu.VMEM((8, h), f32),
            pltpu.VMEM((g, h), f32),
            pltpu.VMEM((g, h), f32),
        ],
    )(acc2, hs2, dinv16, b2.reshape(1, h), g2.reshape(1, h),
      bt2.reshape(1, h), batch3d, esum, ecnt, We2, be2.reshape(1, h))

    return out


# speedup vs baseline: 1.3347x; 1.0130x over previous
import functools

import jax
import jax.numpy as jnp
from jax import lax
from jax.experimental import pallas as pl
from jax.experimental.pallas import tpu as pltpu
from jax.experimental.pallas import tpu_sc as plsc

EPS = 1e-5

NC = 2
NS = 16
NW = NC * NS


def _sc_mesh():
    return plsc.VectorSubcoreMesh(core_axis_name="c", subcore_axis_name="s")


def _make_agg_kernel(npad, e, h, k, ch0_frac=0.5):
    cht = e // (NS * k)
    ch0 = int(round(cht * ch0_frac))
    ch1 = cht - ch0
    rpt = npad // NS
    zr = 64

    @functools.partial(
        pl.kernel,
        mesh=_sc_mesh(),
        out_type=jax.ShapeDtypeStruct((NC, npad, h), jnp.float32),
        scratch_types=[
            pltpu.VMEM((2, k), jnp.int32),
            pltpu.VMEM((2, k), jnp.int32),
            pltpu.VMEM((k, h), jnp.float32),
            pltpu.VMEM((k, h), jnp.float32),
            pltpu.VMEM((zr, h), jnp.float32),
            pltpu.VMEM_SHARED((npad, h), jnp.float32),
            pltpu.SemaphoreType.DMA,
            pltpu.SemaphoreType.DMA,
            pltpu.SemaphoreType.DMA,
            pltpu.SemaphoreType.DMA,
        ],
    )
    def agg_kernel(tab_hbm, idx2_hbm, out_hbm,
                   ibuf0, ibuf1, rows0, rows1, zbuf,
                   acc_sh, gsem0, gsem1, ssem0, ssem1):
        cid = lax.axis_index("c")
        sid = lax.axis_index("s")
        ch = jnp.where(cid == 0, ch0, ch1)
        bch = jnp.where(cid == 0, sid * ch0, NS * ch0 + sid * ch1)

        bufs = ((ibuf0, rows0, gsem0, ssem0),
                (ibuf1, rows1, gsem1, ssem1))

        zero16 = jnp.zeros((16,), jnp.float32)

        def zrow(i, _):
            def lane(j, _):
                zbuf[i, pl.ds(j * 16, 16)] = zero16
                return 0
            lax.fori_loop(0, h // 16, lane, 0)
            return 0

        lax.fori_loop(0, zr, zrow, 0)

        def zcopy(j, _):
            pltpu.sync_copy(zbuf, acc_sh.at[pl.ds(sid * rpt + j * zr, zr)])
            return 0

        lax.fori_loop(0, rpt // zr, zcopy, 0)
        plsc.subcore_barrier()

        def body(t, _):
            for p in (0, 1):
                ibuf, rows, gsem, ssem = bufs[p]
                is_p = t % 2 == p

                @pl.when(is_p & (t >= 2))
                def _():
                    pltpu.make_async_copy(
                        rows, acc_sh.at[ibuf.at[1]], ssem).wait()

                @pl.when(is_p & (t < ch))
                def _():
                    pltpu.sync_copy(idx2_hbm.at[bch + t], ibuf)
                    pltpu.async_copy(tab_hbm.at[ibuf.at[0]], rows, gsem)

                @pl.when(is_p & (t >= 1) & (t <= ch))
                def _():
                    iq, rq, gq, ssq = bufs[1 - p]
                    pltpu.make_async_copy(
                        tab_hbm.at[iq.at[0]], rq, gq).wait()
                    pltpu.async_copy(rq, acc_sh.at[iq.at[1]], ssq, add=True)
            return 0

        lax.fori_loop(0, ch + 2, body, 0)
        plsc.subcore_barrier()
        pltpu.sync_copy(
            acc_sh.at[pl.ds(sid * rpt, rpt)],
            out_hbm.at[cid, pl.ds(sid * rpt, rpt)],
        )

    return agg_kernel


def _edge_deg_body(ea_ref, src_ref, dst_ref, lo_ref, hi_ref, we1_ref, be1_ref,
                   esum_ref, ecnt_ref, dm_ref):
    i = pl.program_id(0)
    bf16 = jnp.bfloat16
    u = jnp.maximum(
        jnp.dot(ea_ref[...], we1_ref[...], preferred_element_type=jnp.float32)
        + be1_ref[...], 0.0)
    srow = src_ref[0]
    lo = lo_ref[:, 0:1]
    hi = hi_ref[:, 0:1]
    oh = ((srow >= lo) & (srow < hi)).astype(bf16)

    drow = dst_ref[0]
    ohhi = (drow // 128
            == lax.broadcasted_iota(jnp.int32, (80, 1), 0)).astype(bf16)
    ohlot = (drow % 128
             == lax.broadcasted_iota(jnp.int32, (128, 1), 0)).astype(bf16)

    @pl.when(i == 0)
    def _():
        esum_ref[...] = jnp.zeros_like(esum_ref)
        ecnt_ref[...] = jnp.zeros_like(ecnt_ref)
        dm_ref[...] = jnp.zeros_like(dm_ref)

    esum_ref[...] += jnp.dot(oh, u.astype(bf16),
                             preferred_element_type=jnp.float32)
    ecnt_ref[...] += jnp.broadcast_to(
        jnp.sum(oh.astype(jnp.float32), axis=1, keepdims=True),
        ecnt_ref.shape)
    dm_ref[...] += lax.dot_general(
        ohhi, ohlot, (((1,), (1,)), ((), ())),
        preferred_element_type=jnp.float32)


def _k1_body(bn, x_ref, w1_ref, dm_ref, hs1_ref, dinv16_ref):
    nsub = bn // 128
    sel = (lax.broadcasted_iota(jnp.int32, (bn, nsub), 0) // 128
           == lax.broadcasted_iota(jnp.int32, (bn, nsub), 1)
           ).astype(jnp.float32)
    expand = lax.dot_general(sel, dm_ref[...], (((1,), (0,)), ((), ())),
                             preferred_element_type=jnp.float32)
    msk = (lax.broadcasted_iota(jnp.int32, (bn, 128), 0) % 128
           == lax.broadcasted_iota(jnp.int32, (bn, 128), 1))
    deg = jnp.sum(jnp.where(msk, expand, 0.0), axis=1, keepdims=True) + 1.0
    dinv = lax.rsqrt(jnp.maximum(deg, 1.0))
    hm = jnp.dot(x_ref[...], w1_ref[...], preferred_element_type=jnp.float32)
    hs1_ref[...] = hm * dinv
    dinv16_ref[...] = jnp.broadcast_to(dinv, dinv16_ref.shape)


def _stats_phase(ngrid, padcnt, bn, accp_ref, hs_ref, dinv16_ref, b_ref,
                 o_scr, stats_ref):
    i = pl.program_id(1)
    dinv = dinv16_ref[:, 0:1]
    o = dinv * (accp_ref[0] + accp_ref[1] + hs_ref[...]) + b_ref[...]
    o_scr[pl.ds(i * bn, bn), :] = o

    @pl.when(i == 0)
    def _():
        stats_ref[...] = jnp.zeros_like(stats_ref)

    stats_ref[0:1, :] += jnp.sum(o, axis=0, keepdims=True)
    stats_ref[1:2, :] += jnp.sum(o * o, axis=0, keepdims=True)

    @pl.when(i == ngrid - 1)
    def _():
        last = o[-1:, :]
        stats_ref[0:1, :] += -float(padcnt) * last
        stats_ref[1:2, :] += -float(padcnt) * last * last


def _bn_from_stats(n, stats_ref):
    mu = stats_ref[0:1, :] * (1.0 / n)
    ex2 = stats_ref[1:2, :] * (1.0 / n)
    inv = lax.rsqrt(ex2 - mu * mu + EPS)
    return mu, inv


def _layer_mid_body(n, bn, ngrid, padcnt,
                    accp_ref, hs_ref, dinv16_ref, b_ref, g_ref, bt_ref,
                    w2_ref, hs2_ref, o_scr, stats_scr):
    j = pl.program_id(0)
    i = pl.program_id(1)

    @pl.when(j == 0)
    def _():
        _stats_phase(ngrid, padcnt, bn, accp_ref, hs_ref, dinv16_ref, b_ref,
                     o_scr, stats_scr)

    @pl.when(j == 1)
    def _():
        mu, inv = _bn_from_stats(n, stats_scr)
        o = o_scr[pl.ds(i * bn, bn), :]
        hcur = jnp.maximum((o - mu) * inv * g_ref[...] + bt_ref[...], 0.0)
        hm = jnp.dot(hcur, w2_ref[...], preferred_element_type=jnp.float32)
        rmask = (lax.broadcasted_iota(jnp.int32, (bn, 1), 0) + i * bn
                 < n).astype(jnp.float32)
        hs2_ref[...] = hm * dinv16_ref[:, 0:1] * rmask


def _layer_final_body(n, g, bn, ngrid, padcnt,
                      accp_ref, hs_ref, dinv16_ref, b_ref, g_ref, bt_ref,
                      batch_ref, esum_ref, ecnt_ref, we2_ref, be2_ref,
                      out_ref, o_scr, stats_scr, nsum_ref, ncnt_ref):
    j = pl.program_id(0)
    i = pl.program_id(1)

    @pl.when(j == 0)
    def _():
        _stats_phase(ngrid, padcnt, bn, accp_ref, hs_ref, dinv16_ref, b_ref,
                     o_scr, stats_scr)

    @pl.when(j == 1)
    def _():
        mu, inv = _bn_from_stats(n, stats_scr)
        o = o_scr[pl.ds(i * bn, bn), :]
        h2 = jnp.maximum((o - mu) * inv * g_ref[...] + bt_ref[...], 0.0)
        brow = batch_ref[0]
        gids = lax.broadcasted_iota(jnp.int32, (g, 1), 0)
        oh = (brow == gids).astype(jnp.float32)

        @pl.when(i == 0)
        def _():
            nsum_ref[...] = jnp.zeros_like(nsum_ref)
            ncnt_ref[...] = jnp.zeros_like(ncnt_ref)

        nsum_ref[...] += jnp.dot(oh, h2, preferred_element_type=jnp.float32)
        ncnt_ref[...] += jnp.broadcast_to(
            jnp.sum(oh, axis=1, keepdims=True), ncnt_ref.shape)

        @pl.when(i == ngrid - 1)
        def _():
            ecnt = ecnt_ref[...]
            edge_part = (
                jnp.dot(esum_ref[...], we2_ref[...],
                        preferred_element_type=jnp.float32)
                + ecnt * be2_ref[...]) / jnp.maximum(ecnt, 1.0)
            node_part = nsum_ref[...] / jnp.maximum(ncnt_ref[...], 1.0)
            out_ref[...] = node_part + edge_part


def kernel(x, edge_index, edge_attr, batch_idx,
           W1, b1, g1, bt1, W2, b2, g2, bt2,
           We1, be1, We2, be2):
    n, df = x.shape
    e = edge_index.shape[1]
    de = edge_attr.shape[1]
    h = W1.shape[1]
    g = 64
    f32 = jnp.float32

    npad = 10240
    padcnt = npad - n
    bn = 2048
    ngrid = npad // bn
    be = 8000
    egrid = e // be

    src = edge_index[0].astype(jnp.int32)
    dst = edge_index[1].astype(jnp.int32)
    src3d = src.reshape(egrid, 1, be)
    dst3d = dst.reshape(egrid, 1, be)

    k_sc = 128
    epad = NW * k_sc * (-(-e // (NW * k_sc)))
    srcp = jnp.full((epad,), npad - 1, jnp.int32).at[:e].set(src)
    dstp = jnp.full((epad,), npad - 1, jnp.int32).at[:e].set(dst)
    idx2 = jnp.stack(
        [srcp.reshape(-1, k_sc), dstp.reshape(-1, k_sc)], axis=1)

    starts = jnp.searchsorted(
        batch_idx.astype(jnp.int32), jnp.arange(g + 1, dtype=jnp.int32)
    ).astype(jnp.int32)
    lo_b = jnp.broadcast_to(starts[:g][:, None], (g, h)).astype(jnp.int32)
    hi_b = jnp.broadcast_to(starts[1:][:, None], (g, h)).astype(jnp.int32)

    xpad = jnp.zeros((npad, df), f32).at[:n].set(x)
    bpad = jnp.full((npad,), g, jnp.int32).at[:n].set(batch_idx.astype(jnp.int32))
    batch3d = bpad.reshape(ngrid, 1, bn)

    esum, ecnt, degmat = pl.pallas_call(
        _edge_deg_body,
        grid=(egrid,),
        in_specs=[
            pl.BlockSpec((be, de), lambda i: (i, 0)),
            pl.BlockSpec((1, 1, be), lambda i: (i, 0, 0)),
            pl.BlockSpec((1, 1, be), lambda i: (i, 0, 0)),
            pl.BlockSpec((g, h), lambda i: (0, 0)),
            pl.BlockSpec((g, h), lambda i: (0, 0)),
            pl.BlockSpec((de, h), lambda i: (0, 0)),
            pl.BlockSpec((1, h), lambda i: (0, 0)),
        ],
        out_specs=[
            pl.BlockSpec((g, h), lambda i: (0, 0)),
            pl.BlockSpec((g, h), lambda i: (0, 0)),
            pl.BlockSpec((80, 128), lambda i: (0, 0)),
        ],
        out_shape=[
            jax.ShapeDtypeStruct((g, h), f32),
            jax.ShapeDtypeStruct((g, h), f32),
            jax.ShapeDtypeStruct((80, 128), f32),
        ],
    )(edge_attr.astype(jnp.bfloat16), src3d, dst3d, lo_b, hi_b,
      We1.astype(jnp.bfloat16), be1.reshape(1, h))

    hs1, dinv16 = pl.pallas_call(
        functools.partial(_k1_body, bn),
        grid=(ngrid,),
        in_specs=[
            pl.BlockSpec((bn, df), lambda i: (i, 0)),
            pl.BlockSpec((df, h), lambda i: (0, 0)),
            pl.BlockSpec((bn // 128, 128), lambda i: (i, 0)),
        ],
        out_specs=[
            pl.BlockSpec((bn, h), lambda i: (i, 0)),
            pl.BlockSpec((bn, 16), lambda i: (i, 0)),
        ],
        out_shape=[
            jax.ShapeDtypeStruct((npad, h), f32),
            jax.ShapeDtypeStruct((npad, 16), f32),
        ],
    )(xpad, W1, degmat)

    agg = _make_agg_kernel(npad, epad, h, k_sc, ch0_frac=0.875)

    acc1 = agg(hs1, idx2)
    hs2 = pl.pallas_call(
        functools.partial(_layer_mid_body, n, bn, ngrid, padcnt),
        grid=(2, ngrid),
        in_specs=[
            pl.BlockSpec((NC, bn, h), lambda j, i: (0, i * (1 - j), 0)),
            pl.BlockSpec((bn, h), lambda j, i: (i * (1 - j), 0)),
            pl.BlockSpec((bn, 16), lambda j, i: (i, 0)),
            pl.BlockSpec((1, h), lambda j, i: (0, 0)),
            pl.BlockSpec((1, h), lambda j, i: (0, 0)),
            pl.BlockSpec((1, h), lambda j, i: (0, 0)),
            pl.BlockSpec((h, h), lambda j, i: (0, 0)),
        ],
        out_specs=pl.BlockSpec((bn, h), lambda j, i: (i * j, 0)),
        out_shape=jax.ShapeDtypeStruct((npad, h), f32),
        scratch_shapes=[
            pltpu.VMEM((npad, h), f32),
            pltpu.VMEM((8, h), f32),
        ],
    )(acc1, hs1, dinv16, b1.reshape(1, h), g1.reshape(1, h),
      bt1.reshape(1, h), W2)

    acc2 = agg(hs2, idx2)
    out = pl.pallas_call(
        functools.partial(_layer_final_body, n, g, bn, ngrid, padcnt),
        grid=(2, ngrid),
        in_specs=[
            pl.BlockSpec((NC, bn, h), lambda j, i: (0, i * (1 - j), 0)),
            pl.BlockSpec((bn, h), lambda j, i: (i * (1 - j), 0)),
            pl.BlockSpec((bn, 16), lambda j, i: (i, 0)),
            pl.BlockSpec((1, h), lambda j, i: (0, 0)),
            pl.BlockSpec((1, h), lambda j, i: (0, 0)),
            pl.BlockSpec((1, h), lambda j, i: (0, 0)),
            pl.BlockSpec((1, 1, bn), lambda j, i: (i, 0, 0)),
            pl.BlockSpec((g, h), lambda j, i: (0, 0)),
            pl.BlockSpec((g, h), lambda j, i: (0, 0)),
            pl.BlockSpec((h, h), lambda j, i: (0, 0)),
            pl.BlockSpec((1, h), lambda j, i: (0, 0)),
        ],
        out_specs=pl.BlockSpec((g, h), lambda j, i: (0, 0)),
        out_shape=jax.ShapeDtypeStruct((g, h), f32),
        scratch_shapes=[
            pltpu.VMEM((npad, h), f32),
            pltpu.VMEM((8, h), f32),
            pltpu.VMEM((g, h), f32),
            pltpu.VMEM((g, h), f32),
        ],
    )(acc2, hs2, dinv16, b2.reshape(1, h), g2.reshape(1, h),
      bt2.reshape(1, h), batch3d, esum, ecnt, We2, be2.reshape(1, h))

    return out
